# baseline (speedup 1.0000x reference)
import functools

import jax
import jax.numpy as jnp
from jax import lax
from jax.experimental import pallas as pl
from jax.experimental.pallas import tpu as pltpu
from jax.experimental.pallas import tpu_sc as plsc

_N = 10000
_E = 320000
_DF = 128
_DH = 16
_DL2 = 16
_NCLS = 3

_NCORES = 2
_NSUB = 16
_NW = _NCORES * _NSUB
_CHUNK = 128
_EPT = _E // _NW
_NCHUNK = -(-_EPT // _CHUNK)
_EPT_PAD = _NCHUNK * _CHUNK
_RPT = 640
_NPAD = _RPT * _NSUB

_NBUF = 8
_D = 4
_NGRP = -(-_NCHUNK // _NBUF)

_BM = 2000
_GRID = _N // _BM


def _proj1_body(x_ref, wl_ref, wr_ref, b_ref, y_ref, r_ref):
    xb = x_ref[...]
    y_ref[...] = jnp.dot(xb, wl_ref[...], preferred_element_type=jnp.float32)
    r_ref[...] = (jnp.dot(xb, wr_ref[...], preferred_element_type=jnp.float32)
                  + b_ref[...])


def _mid_body(s_ref, d_ref, r_ref, wl_ref, wr_ref, b_ref, y_ref, r2_ref):
    s = s_ref[0] + s_ref[1]
    d = jnp.maximum(d_ref[0] + d_ref[1], 1.0)
    h = jnp.maximum(s / d + r_ref[...], 0.0)
    y_ref[...] = jnp.dot(h, wl_ref[...], preferred_element_type=jnp.float32)
    r2_ref[...] = (jnp.dot(h, wr_ref[...], preferred_element_type=jnp.float32)
                   + b_ref[...])


def _out_body(s_ref, d_ref, r_ref, o_ref):
    s = s_ref[0] + s_ref[1]
    d = jnp.maximum(d_ref[0] + d_ref[1], 1.0)
    z = jnp.maximum(s / d + r_ref[...], 0.0)
    col = lax.broadcasted_iota(jnp.int32, z.shape, 1)
    mask = col < _NCLS
    zm = jnp.where(mask, z, -1e30)
    m = jnp.max(zm, axis=1, keepdims=True)
    e = jnp.where(mask, jnp.exp(z - m), 0.0)
    se = jnp.sum(e, axis=1, keepdims=True)
    o_ref[...] = (z - m - jnp.log(se))[:, :_NCLS]


_proj1 = pl.pallas_call(
    _proj1_body,
    grid=(_GRID,),
    in_specs=[
        pl.BlockSpec((_BM, _DF), lambda i: (i, 0)),
        pl.BlockSpec((_DF, _DH), lambda i: (0, 0)),
        pl.BlockSpec((_DF, _DH), lambda i: (0, 0)),
        pl.BlockSpec((1, _DH), lambda i: (0, 0)),
    ],
    out_specs=[
        pl.BlockSpec((_BM, _DH), lambda i: (i, 0)),
        pl.BlockSpec((_BM, _DH), lambda i: (i, 0)),
    ],
    out_shape=[
        jax.ShapeDtypeStruct((_N, _DH), jnp.float32),
        jax.ShapeDtypeStruct((_N, _DH), jnp.float32),
    ],
)

_mid = pl.pallas_call(
    _mid_body,
    grid=(_GRID,),
    in_specs=[
        pl.BlockSpec((_NCORES, _BM, _DH), lambda i: (0, i, 0)),
        pl.BlockSpec((_NCORES, _BM, _DH), lambda i: (0, i, 0)),
        pl.BlockSpec((_BM, _DH), lambda i: (i, 0)),
        pl.BlockSpec((_DH, _DL2), lambda i: (0, 0)),
        pl.BlockSpec((_DH, _DL2), lambda i: (0, 0)),
        pl.BlockSpec((1, _DL2), lambda i: (0, 0)),
    ],
    out_specs=[
        pl.BlockSpec((_BM, _DL2), lambda i: (i, 0)),
        pl.BlockSpec((_BM, _DL2), lambda i: (i, 0)),
    ],
    out_shape=[
        jax.ShapeDtypeStruct((_N, _DL2), jnp.float32),
        jax.ShapeDtypeStruct((_N, _DL2), jnp.float32),
    ],
)

_outk = pl.pallas_call(
    _out_body,
    grid=(_GRID,),
    in_specs=[
        pl.BlockSpec((_NCORES, _BM, _DL2), lambda i: (0, i, 0)),
        pl.BlockSpec((_NCORES, _BM, _DH), lambda i: (0, i, 0)),
        pl.BlockSpec((_BM, _DL2), lambda i: (i, 0)),
    ],
    out_specs=pl.BlockSpec((_BM, _NCLS), lambda i: (i, 0)),
    out_shape=jax.ShapeDtypeStruct((_N, _NCLS), jnp.float32),
)


_mesh = plsc.VectorSubcoreMesh(core_axis_name="c", subcore_axis_name="s",
                               num_cores=_NCORES, num_subcores=_NSUB)


def _make_seg_kernel(width: int, with_deg: bool):
    acc_t = jax.ShapeDtypeStruct((_NCORES, _NPAD, width), jnp.float32)
    deg_t = jax.ShapeDtypeStruct((_NCORES, _NPAD, _DH), jnp.float32)
    out_type = [acc_t, deg_t] if with_deg else acc_t
    scratch = [
        pltpu.VMEM((_NCHUNK, _CHUNK), jnp.int32),
        pltpu.VMEM((_NCHUNK, _CHUNK), jnp.int32),
        pltpu.VMEM((_NBUF, _CHUNK, width), jnp.float32),
        pltpu.VMEM_SHARED((_NPAD, width), jnp.float32),
        pltpu.SemaphoreType.DMA((_NBUF,)),
        pltpu.SemaphoreType.DMA((_NBUF,)),
    ]
    if with_deg:
        scratch.insert(3, pltpu.VMEM((_CHUNK, _DH), jnp.float32))
        scratch.insert(5, pltpu.VMEM_SHARED((_NPAD, _DH), jnp.float32))
        scratch.append(pltpu.SemaphoreType.DMA((_NBUF,)))

    def body(vals_hbm, src_hbm, dst_hbm, zeros_hbm, *rest):
        if with_deg:
            (ones_hbm, out_hbm, deg_hbm, src_v, dst_v, rows_v,
             ones_v, acc_sh, deg_sh, gsem, ssem, osem) = rest
        else:
            (out_hbm, src_v, dst_v, rows_v, acc_sh, gsem, ssem) = rest
        cid = lax.axis_index("c")
        sid = lax.axis_index("s")
        wid = cid * _NSUB + sid
        pltpu.sync_copy(src_hbm.at[wid], src_v)
        pltpu.sync_copy(dst_hbm.at[wid], dst_v)
        if with_deg:
            pltpu.sync_copy(ones_hbm, ones_v)

        def fire_gather(j, b):
            pltpu.async_copy(vals_hbm.at[src_v.at[j]], rows_v.at[b],
                             gsem.at[b])

        def wait_gather(j, b):
            pltpu.make_async_copy(vals_hbm.at[src_v.at[j]], rows_v.at[b],
                                  gsem.at[b]).wait()

        def fire_scatter(j, b):
            pltpu.async_copy(rows_v.at[b], acc_sh.at[dst_v.at[j]],
                             ssem.at[b], add=True)
            if with_deg:
                pltpu.async_copy(ones_v, deg_sh.at[dst_v.at[j]],
                                 osem.at[b], add=True)

        def wait_scatter(j, b):
            pltpu.make_async_copy(rows_v.at[b], acc_sh.at[dst_v.at[j]],
                                  ssem.at[b]).wait()
            if with_deg:
                pltpu.make_async_copy(ones_v, deg_sh.at[dst_v.at[j]],
                                      osem.at[b]).wait()

        for b in range(_D):
            fire_gather(b, b)
        r0 = sid * _RPT
        pltpu.sync_copy(zeros_hbm.at[pl.ds(r0, _RPT)],
                        acc_sh.at[pl.ds(r0, _RPT)])
        if with_deg:
            pltpu.sync_copy(zeros_hbm.at[pl.ds(r0, _RPT)],
                            deg_sh.at[pl.ds(r0, _RPT)])
        plsc.subcore_barrier()

        def group(gi, carry):
            for b in range(_NBUF):
                j = gi * _NBUF + b

                @pl.when(j < _NCHUNK)
                def _turn():
                    @pl.when(j >= _D)
                    def _():
                        wait_scatter(j - _D, (b + _D) % _NBUF)

                    @pl.when(j + _D < _NCHUNK)
                    def _():
                        fire_gather(j + _D, (b + _D) % _NBUF)

                    wait_gather(j, b)
                    fire_scatter(j, b)
            return carry

        lax.fori_loop(0, _NGRP, group, 0)
        for j in range(_NCHUNK - _D, _NCHUNK):
            wait_scatter(j, j % _NBUF)
        plsc.subcore_barrier()
        pltpu.sync_copy(acc_sh.at[pl.ds(r0, _RPT)],
                        out_hbm.at[cid, pl.ds(r0, _RPT)])
        if with_deg:
            pltpu.sync_copy(deg_sh.at[pl.ds(r0, _RPT)],
                            deg_hbm.at[cid, pl.ds(r0, _RPT)])

    return functools.partial(
        pl.kernel, mesh=_mesh, out_type=out_type, scratch_types=scratch,
        compiler_params=pltpu.CompilerParams(use_tc_tiling_on_sc=False),
    )(body)


_seg_deg = _make_seg_kernel(_DH, with_deg=True)
_seg = _make_seg_kernel(_DL2, with_deg=False)


def kernel(x, edge_index, W1_l, b1, W1_r, W2_l, b2, W2_r):
    f32 = jnp.float32
    src = edge_index[0].reshape(_NW, _EPT)
    dst = edge_index[1].reshape(_NW, _EPT)
    pad = _EPT_PAD - _EPT
    src = jnp.pad(src, ((0, 0), (0, pad))).reshape(_NW, _NCHUNK, _CHUNK)
    dst = jnp.pad(dst, ((0, 0), (0, pad)), constant_values=_N)
    dst = dst.reshape(_NW, _NCHUNK, _CHUNK)

    zeros16 = jnp.zeros((_NPAD, _DH), f32)
    ones = jnp.ones((_CHUNK, _DH), f32)

    y1, r1 = _proj1(x, W1_l, W1_r, b1.reshape(1, _DH))
    s1, deg = _seg_deg(y1, src, dst, zeros16, ones)

    w2l = jnp.pad(W2_l, ((0, 0), (0, _DL2 - _NCLS)))
    w2r = jnp.pad(W2_r, ((0, 0), (0, _DL2 - _NCLS)))
    b2p = jnp.pad(b2, (0, _DL2 - _NCLS)).reshape(1, _DL2)
    y2, r2 = _mid(s1, deg, r1, w2l, w2r, b2p)

    s2 = _seg(y2, src, dst, zeros16)
    return _outk(s2, deg, r2)


# speedup vs baseline: 1.0882x; 1.0882x over previous
import functools

import jax
import jax.numpy as jnp
from jax import lax
from jax.experimental import pallas as pl
from jax.experimental.pallas import tpu as pltpu
from jax.experimental.pallas import tpu_sc as plsc

_N = 10000
_E = 320000
_DF = 128
_DH = 16
_NCLS = 3

_NCORES = 2
_NSUB = 16
_NW = _NCORES * _NSUB
_CHUNK = 128
_EPT = _E // _NW
_NCHUNK = -(-_EPT // _CHUNK)
_EPT_PAD = _NCHUNK * _CHUNK
_RPT = 640
_NPAD = _RPT * _NSUB
_RPW = _NPAD // _NW

_NBUF = 8
_D = 4
_NGRP = -(-_NCHUNK // _NBUF)

_BM = 2000
_GRID = _N // _BM


def _proj1_body(x_ref, wl_ref, wr_ref, b_ref, y_ref, r_ref):
    xb = x_ref[...]
    y_ref[...] = jnp.dot(xb, wl_ref[...], preferred_element_type=jnp.float32)
    r_ref[...] = (jnp.dot(xb, wr_ref[...], preferred_element_type=jnp.float32)
                  + b_ref[...])


def _out_body(t_ref, d_ref, h_ref, wl_ref, wr_ref, b_ref, o_ref):
    agg = (t_ref[0] + t_ref[1]) / d_ref[...]
    h = h_ref[...]
    z = (jnp.dot(agg, wl_ref[...], preferred_element_type=jnp.float32)
         + jnp.dot(h, wr_ref[...], preferred_element_type=jnp.float32)
         + b_ref[...])
    z = jnp.maximum(z, 0.0)
    m = jnp.max(z, axis=1, keepdims=True)
    se = jnp.sum(jnp.exp(z - m), axis=1, keepdims=True)
    o_ref[...] = z - m - jnp.log(se)


_proj1 = pl.pallas_call(
    _proj1_body,
    grid=(_GRID,),
    in_specs=[
        pl.BlockSpec((_BM, _DF), lambda i: (i, 0)),
        pl.BlockSpec((_DF, _DH), lambda i: (0, 0)),
        pl.BlockSpec((_DF, _DH), lambda i: (0, 0)),
        pl.BlockSpec((1, _DH), lambda i: (0, 0)),
    ],
    out_specs=[
        pl.BlockSpec((_BM, _DH), lambda i: (i, 0)),
        pl.BlockSpec((_BM, _DH), lambda i: (i, 0)),
    ],
    out_shape=[
        jax.ShapeDtypeStruct((_N, _DH), jnp.float32),
        jax.ShapeDtypeStruct((_NPAD, _DH), jnp.float32),
    ],
)

_outk = pl.pallas_call(
    _out_body,
    grid=(_GRID,),
    in_specs=[
        pl.BlockSpec((_NCORES, _BM, _DH), lambda i: (0, i, 0)),
        pl.BlockSpec((_BM, _DH), lambda i: (i, 0)),
        pl.BlockSpec((_BM, _DH), lambda i: (i, 0)),
        pl.BlockSpec((_DH, _NCLS), lambda i: (0, 0)),
        pl.BlockSpec((_DH, _NCLS), lambda i: (0, 0)),
        pl.BlockSpec((1, _NCLS), lambda i: (0, 0)),
    ],
    out_specs=pl.BlockSpec((_BM, _NCLS), lambda i: (i, 0)),
    out_shape=jax.ShapeDtypeStruct((_N, _NCLS), jnp.float32),
)


_mesh = plsc.VectorSubcoreMesh(core_axis_name="c", subcore_axis_name="s",
                               num_cores=_NCORES, num_subcores=_NSUB)

_acc_t = jax.ShapeDtypeStruct((_NCORES, _NPAD, _DH), jnp.float32)
_vec_t = jax.ShapeDtypeStruct((_NPAD, _DH), jnp.float32)


@functools.partial(
    pl.kernel,
    mesh=_mesh,
    out_type=[_acc_t, _vec_t, _vec_t, _acc_t, _acc_t],
    scratch_types=[
        pltpu.VMEM((_NCHUNK, _CHUNK), jnp.int32),
        pltpu.VMEM((_NCHUNK, _CHUNK), jnp.int32),
        pltpu.VMEM((_NBUF, _CHUNK, _DH), jnp.float32),
        pltpu.VMEM((_CHUNK, _DH), jnp.float32),
        pltpu.VMEM((_RPW, _DH), jnp.float32),
        pltpu.VMEM((_RPW, _DH), jnp.float32),
        pltpu.VMEM((_RPW, _DH), jnp.float32),
        pltpu.VMEM((_RPW, _DH), jnp.float32),
        pltpu.VMEM((_RPW, _DH), jnp.float32),
        pltpu.VMEM((_RPW, _DH), jnp.float32),
        pltpu.VMEM((_RPW, _DH), jnp.float32),
        pltpu.VMEM_SHARED((_NPAD, _DH), jnp.float32),
        pltpu.VMEM_SHARED((_NPAD, _DH), jnp.float32),
        pltpu.SemaphoreType.DMA((_NBUF,)),
        pltpu.SemaphoreType.DMA((_NBUF,)),
        pltpu.SemaphoreType.DMA((_NBUF,)),
        pltpu.SemaphoreType.REGULAR,
    ],
    compiler_params=pltpu.CompilerParams(use_tc_tiling_on_sc=False),
)
def _seg_fused(y1_hbm, r1_hbm, src_hbm, dst_hbm, zeros_hbm, ones_hbm,
               t2_hbm, dt_hbm, h_hbm, xa_hbm, xd_hbm,
               src_v, dst_v, rows_v, ones_v, a0_v, a1_v, d0_v, d1_v, r1_v,
               h_v, dt_v, acc_sh, deg_sh, gsem, ssem, osem, xsem):
    cid = lax.axis_index("c")
    sid = lax.axis_index("s")
    wid = cid * _NSUB + sid
    pltpu.sync_copy(src_hbm.at[wid], src_v)
    pltpu.sync_copy(dst_hbm.at[wid], dst_v)
    pltpu.sync_copy(ones_hbm, ones_v)

    def ring_pass(vals, acc, deg):
        def fire_gather(j, b):
            pltpu.async_copy(vals.at[src_v.at[j]], rows_v.at[b], gsem.at[b])

        def wait_gather(j, b):
            pltpu.make_async_copy(vals.at[src_v.at[j]], rows_v.at[b],
                                  gsem.at[b]).wait()

        def fire_scatter(j, b):
            pltpu.async_copy(rows_v.at[b], acc.at[dst_v.at[j]],
                             ssem.at[b], add=True)
            if deg is not None:
                pltpu.async_copy(ones_v, deg.at[dst_v.at[j]],
                                 osem.at[b], add=True)

        def wait_scatter(j, b):
            pltpu.make_async_copy(rows_v.at[b], acc.at[dst_v.at[j]],
                                  ssem.at[b]).wait()
            if deg is not None:
                pltpu.make_async_copy(ones_v, deg.at[dst_v.at[j]],
                                      osem.at[b]).wait()

        for b in range(_D):
            fire_gather(b, b)

        def group(gi, carry):
            for b in range(_NBUF):
                j = gi * _NBUF + b

                @pl.when(j < _NCHUNK)
                def _turn():
                    @pl.when(j >= _D)
                    def _():
                        wait_scatter(j - _D, (b + _D) % _NBUF)

                    @pl.when(j + _D < _NCHUNK)
                    def _():
                        fire_gather(j + _D, (b + _D) % _NBUF)

                    wait_gather(j, b)
                    fire_scatter(j, b)
            return carry

        lax.fori_loop(0, _NGRP, group, 0)
        for j in range(_NCHUNK - _D, _NCHUNK):
            wait_scatter(j, j % _NBUF)

    def handshake():
        plsc.subcore_barrier()

        @pl.when(sid == 0)
        def _():
            pl.semaphore_signal(xsem, 1, core_index=1 - cid)
            pl.semaphore_wait(xsem, 1)

        plsc.subcore_barrier()

    r0 = sid * _RPT
    slab = pl.ds(r0, _RPT)
    pltpu.sync_copy(zeros_hbm.at[pl.ds(0, _RPT)], acc_sh.at[slab])
    pltpu.sync_copy(zeros_hbm.at[pl.ds(0, _RPT)], deg_sh.at[slab])
    plsc.subcore_barrier()
    ring_pass(y1_hbm, acc_sh, deg_sh)
    plsc.subcore_barrier()

    pltpu.sync_copy(acc_sh.at[slab], xa_hbm.at[cid, slab])
    pltpu.sync_copy(deg_sh.at[slab], xd_hbm.at[cid, slab])
    handshake()

    g0 = wid * _RPW
    gslab = pl.ds(g0, _RPW)
    pltpu.sync_copy(xa_hbm.at[0, gslab], a0_v)
    pltpu.sync_copy(xa_hbm.at[1, gslab], a1_v)
    pltpu.sync_copy(xd_hbm.at[0, gslab], d0_v)
    pltpu.sync_copy(xd_hbm.at[1, gslab], d1_v)
    pltpu.sync_copy(r1_hbm.at[gslab], r1_v)

    def hrow(i, carry):
        d = jnp.maximum(d0_v[i] + d1_v[i], 1.0)
        h = jnp.maximum((a0_v[i] + a1_v[i]) / d + r1_v[i], 0.0)
        h_v[i] = h
        dt_v[i] = d
        return carry

    lax.fori_loop(0, _RPW, hrow, 0)
    pltpu.sync_copy(h_v, h_hbm.at[gslab])
    pltpu.sync_copy(dt_v, dt_hbm.at[gslab])
    pltpu.sync_copy(zeros_hbm.at[pl.ds(0, _RPT)], acc_sh.at[slab])
    handshake()

    ring_pass(h_hbm, acc_sh, None)
    plsc.subcore_barrier()
    pltpu.sync_copy(acc_sh.at[slab], t2_hbm.at[cid, slab])


def kernel(x, edge_index, W1_l, b1, W1_r, W2_l, b2, W2_r):
    f32 = jnp.float32
    src = edge_index[0].reshape(_NW, _EPT)
    dst = edge_index[1].reshape(_NW, _EPT)
    pad = _EPT_PAD - _EPT
    src = jnp.pad(src, ((0, 0), (0, pad))).reshape(_NW, _NCHUNK, _CHUNK)
    dst = jnp.pad(dst, ((0, 0), (0, pad)), constant_values=_N)
    dst = dst.reshape(_NW, _NCHUNK, _CHUNK)

    zeros = jnp.zeros((_RPT, _DH), f32)
    ones = jnp.ones((_CHUNK, _DH), f32)

    y1, r1 = _proj1(x, W1_l, W1_r, b1.reshape(1, _DH))
    t2, dtot, h, _, _ = _seg_fused(y1, r1, src, dst, zeros, ones)
    return _outk(t2, dtot, h, W2_l, W2_r, b2.reshape(1, _NCLS))
